# baseline (speedup 1.0000x reference)
import jax
import jax.numpy as jnp
from jax.experimental import pallas as pl
from jax.experimental.pallas import tpu as pltpu

_CHUNKS = (1, 1, 2, 4, 8, 8, 8, 8, 8, 8, 4, 2, 1, 1)
_N = len(_CHUNKS)
_B = 6
_D = 3
_OFFS = tuple(sum(_CHUNKS[:i]) for i in range(_N))


def _dma_pipeline(x_ref, o_ref, buf, sin, sout):
    def cp_in(i):
        return pltpu.make_async_copy(
            x_ref.at[pl.ds(_OFFS[i], _CHUNKS[i])],
            buf.at[i % _B, pl.ds(0, _CHUNKS[i])], sin.at[i])

    def cp_out(i):
        return pltpu.make_async_copy(
            buf.at[i % _B, pl.ds(0, _CHUNKS[i])],
            o_ref.at[pl.ds(_OFFS[i], _CHUNKS[i])], sout.at[i])

    for j in range(_D):
        cp_in(j).start()
    for i in range(_N):
        cp_in(i).wait()
        cp_out(i).start()
        j = i + _D
        if j < _N:
            if j - _B >= 0:
                cp_out(j - _B).wait()
            cp_in(j).start()
    for i in range(_N - _B, _N):
        cp_out(i).wait()


def kernel(x):
    b, c, f = x.shape
    return pl.pallas_call(
        _dma_pipeline,
        in_specs=[pl.BlockSpec(memory_space=pl.ANY)],
        out_specs=pl.BlockSpec(memory_space=pl.ANY),
        out_shape=jax.ShapeDtypeStruct((b, c, f), x.dtype),
        scratch_shapes=[
            pltpu.VMEM((_B, max(_CHUNKS), c, f), x.dtype),
            pltpu.SemaphoreType.DMA((_N,)),
            pltpu.SemaphoreType.DMA((_N,)),
        ],
    )(x)


# speedup vs baseline: 1.0037x; 1.0037x over previous
import jax
import jax.numpy as jnp
from jax.experimental import pallas as pl
from jax.experimental.pallas import tpu as pltpu

_CHUNKS = (2, 2, 4, 8, 8, 8, 8, 8, 8, 4, 2, 2)
_N = len(_CHUNKS)
_B = 6
_D = 3
_OFFS = tuple(sum(_CHUNKS[:i]) for i in range(_N))


def _dma_pipeline(x_ref, o_ref, buf, sin, sout):
    def cp_in(i):
        return pltpu.make_async_copy(
            x_ref.at[pl.ds(_OFFS[i], _CHUNKS[i])],
            buf.at[i % _B, pl.ds(0, _CHUNKS[i])], sin.at[i])

    def cp_out(i):
        return pltpu.make_async_copy(
            buf.at[i % _B, pl.ds(0, _CHUNKS[i])],
            o_ref.at[pl.ds(_OFFS[i], _CHUNKS[i])], sout.at[i])

    for j in range(_D):
        cp_in(j).start()
    for i in range(_N):
        cp_in(i).wait()
        cp_out(i).start()
        j = i + _D
        if j < _N:
            if j - _B >= 0:
                cp_out(j - _B).wait()
            cp_in(j).start()
    for i in range(_N - _B, _N):
        cp_out(i).wait()


def kernel(x):
    b, c, f = x.shape
    return pl.pallas_call(
        _dma_pipeline,
        in_specs=[pl.BlockSpec(memory_space=pl.ANY)],
        out_specs=pl.BlockSpec(memory_space=pl.ANY),
        out_shape=jax.ShapeDtypeStruct((b, c, f), x.dtype),
        scratch_shapes=[
            pltpu.VMEM((_B, max(_CHUNKS), c, f), x.dtype),
            pltpu.SemaphoreType.DMA((_N,)),
            pltpu.SemaphoreType.DMA((_N,)),
        ],
    )(x)


# speedup vs baseline: 1.0066x; 1.0029x over previous
import jax
from jax.experimental import pallas as pl
from jax.experimental.pallas import tpu as pltpu

_CHUNKS = (2, 2, 4, 8, 8, 8, 8, 8, 8, 4, 2, 2)
_N = len(_CHUNKS)
_B = 6
_D = 3
_OFFS = tuple(sum(_CHUNKS[:i]) for i in range(_N))


def _dma_pipeline(x_ref, o_ref, buf, sin, sout):
    def cp_in(i):
        return pltpu.make_async_copy(
            x_ref.at[pl.ds(_OFFS[i], _CHUNKS[i])],
            buf.at[i % _B, pl.ds(0, _CHUNKS[i])], sin.at[i])

    def cp_out(i):
        return pltpu.make_async_copy(
            buf.at[i % _B, pl.ds(0, _CHUNKS[i])],
            o_ref.at[pl.ds(_OFFS[i], _CHUNKS[i])], sout.at[i])

    for j in range(_D):
        cp_in(j).start()
    for i in range(_N):
        cp_in(i).wait()
        cp_out(i).start()
        j = i + _D
        if j < _N:
            if j - _B >= 0:
                cp_out(j - _B).wait()
            cp_in(j).start()
    for i in range(_N - _B, _N):
        cp_out(i).wait()


def kernel(x):
    b, c, f = x.shape
    return pl.pallas_call(
        _dma_pipeline,
        in_specs=[pl.BlockSpec(memory_space=pl.ANY)],
        out_specs=pl.BlockSpec(memory_space=pl.ANY),
        out_shape=jax.ShapeDtypeStruct((b, c, f), x.dtype),
        scratch_shapes=[
            pltpu.VMEM((_B, max(_CHUNKS), c, f), x.dtype),
            pltpu.SemaphoreType.DMA((_N,)),
            pltpu.SemaphoreType.DMA((_N,)),
        ],
    )(x)
